# baseline (speedup 1.0000x reference)
import jax
import jax.numpy as jnp
from jax import lax
from jax.experimental import pallas as pl
from jax.experimental.pallas import tpu as pltpu
from jax.experimental.pallas import tpu_sc as plsc

_B = 16384
_D = 128
_NC = 2
_NS = 16
_NW = _NC * _NS
_PER_W = _B // _NW
_C = 16
_NSUB = _PER_W // _C
_WT = 5.0


def _body(E_re, E_im, E2_re, E2_im, Rp, Rsp, Rop, Tsp, Top,
          sov, rv, tv, out,
          idx_so, idx_r, idx_t, erows, srows, out_v, sems):
  cid = lax.axis_index("c")
  sid = lax.axis_index("s")
  wid = sid * _NC + cid
  base = wid * _PER_W

  pltpu.sync_copy(sov.at[pl.ds(base * 2, _PER_W * 2)], idx_so)
  pltpu.sync_copy(rv.at[pl.ds(base, _PER_W)], idx_r)
  pltpu.sync_copy(tv.at[pl.ds(base, _PER_W)], idx_t)

  egathers = (E_re, E_im, E2_re, E2_im)
  sgathers = (
      (Rp, idx_r),
      (Rsp, idx_r),
      (Rop, idx_r),
      (Tsp, idx_t),
      (Top, idx_t),
  )

  lanes = lax.iota(jnp.int32, 16)

  def fire(sub, b):
    sl2 = pl.ds(sub * _C * 2, _C * 2)
    sl = pl.ds(sub * _C, _C)
    for k, tbl in enumerate(egathers):
      pltpu.async_copy(tbl.at[idx_so.at[sl2]], erows.at[b, k], sems.at[b])
    for j, (tbl, idxf) in enumerate(sgathers):
      pltpu.async_copy(tbl.at[idxf.at[sl]], srows.at[b, j], sems.at[b])

  def wait(sub, b):
    sl2 = pl.ds(sub * _C * 2, _C * 2)
    sl = pl.ds(sub * _C, _C)
    for k, tbl in enumerate(egathers):
      pltpu.make_async_copy(tbl.at[idx_so.at[sl2]], erows.at[b, k],
                            sems.at[b]).wait()
    for j, (tbl, idxf) in enumerate(sgathers):
      pltpu.make_async_copy(tbl.at[idxf.at[sl]], srows.at[b, j],
                            sems.at[b]).wait()

  def compute(sub, b):
    def dbody(d, accs):
      a0, a1 = accs
      col = jnp.bitwise_and(d + lanes, _D - 1)

      def lde(k, off):
        return plsc.load_gather(erows.at[b, k], [off + lanes, col])

      def lds(j):
        xi = plsc.load_gather(srows.at[b, j], [lanes, col])
        xb = plsc.bitcast(xi, jnp.bfloat16)
        return plsc.unpack(xb, format=plsc.PackFormat.INTERLEAVED)

      esr, esi = lde(0, 0), lde(1, 0)
      eor, eoi = lde(0, 16), lde(1, 16)
      e2sr, e2si = lde(2, 0), lde(3, 0)
      e2or, e2oi = lde(2, 16), lde(3, 16)
      rrr, rri = lds(0)
      rsr, rsi = lds(1)
      ror, roi = lds(2)
      tsr, tsi = lds(3)
      tor, toi = lds(4)
      bre = esr * rrr - esi * rri
      bim = esr * rri + esi * rrr
      term = bre * eor + bim * eoi
      sre = esr * rsr - esi * rsi
      sim = esr * rsi + esi * rsr
      srt = sre * tsr + sim * tsi
      ore = eor * ror - eoi * roi
      oim = eor * roi + eoi * ror
      ort = ore * tor + oim * toi
      tre = e2sr * tsr - e2si * tsi
      tim = e2sr * tsi + e2si * tsr
      sot = tre * e2or + tim * e2oi
      return (a0 + (term + srt), a1 + (ort + _WT * sot))

    z = jnp.zeros((16,), jnp.float32)
    a0, a1 = lax.fori_loop(0, _D, dbody, (z, z), unroll=2)
    out_v[pl.ds(sub * _C, _C)] = a0 + a1

  fire(0, 0)

  def pipe(it, carry):
    sub0 = it * 2
    fire(sub0 + 1, 1)
    wait(sub0, 0)
    compute(sub0, 0)

    @pl.when(sub0 + 2 < _NSUB)
    def _():
      fire(sub0 + 2, 0)

    wait(sub0 + 1, 1)
    compute(sub0 + 1, 1)
    return carry

  lax.fori_loop(0, _NSUB // 2, pipe, 0)
  pltpu.sync_copy(out_v, out.at[pl.ds(base, _PER_W)])


_mesh = plsc.VectorSubcoreMesh(core_axis_name="c", subcore_axis_name="s",
                               num_cores=_NC, num_subcores=_NS)

_score = pl.kernel(
    _body,
    out_type=jax.ShapeDtypeStruct((_B,), jnp.float32),
    mesh=_mesh,
    scratch_types=[
        pltpu.VMEM((_PER_W * 2,), jnp.int32),
        pltpu.VMEM((_PER_W,), jnp.int32),
        pltpu.VMEM((_PER_W,), jnp.int32),
        pltpu.VMEM((2, 4, 2 * _C, _D), jnp.float32),
        pltpu.VMEM((2, 5, _C, _D), jnp.int32),
        pltpu.VMEM((_PER_W,), jnp.float32),
        pltpu.SemaphoreType.DMA((2,)),
    ],
    compiler_params=pltpu.CompilerParams(needs_layout_passes=False),
)


def _interleave_so(s, o):
  return jnp.concatenate(
      [s.reshape(-1, _C), o.reshape(-1, _C)], axis=1).reshape(-1)


def _pack_bf16_pair(re, im):
  n = re.shape[0]
  both = jnp.stack([re.astype(jnp.bfloat16), im.astype(jnp.bfloat16)],
                   axis=-1)
  return lax.bitcast_convert_type(both, jnp.int32).reshape(n, _D)


@jax.jit
def kernel(E_re, E_im, E2_re, E2_im, R_re, R_im, Rs_re, Rs_im, Ro_re, Ro_im,
           Ts_re, Ts_im, To_re, To_im, s, r, o, t):
  return _score(E_re, E_im, E2_re, E2_im,
                _pack_bf16_pair(R_re, R_im),
                _pack_bf16_pair(_WT * Rs_re, _WT * Rs_im),
                _pack_bf16_pair(_WT * Ro_re, _WT * Ro_im),
                _pack_bf16_pair(Ts_re, Ts_im),
                _pack_bf16_pair(To_re, To_im),
                _interleave_so(s.astype(jnp.int32), o.astype(jnp.int32)),
                r.astype(jnp.int32), t.astype(jnp.int32))


# speedup vs baseline: 1.0395x; 1.0395x over previous
import jax
import jax.numpy as jnp
from jax import lax
from jax.experimental import pallas as pl
from jax.experimental.pallas import tpu as pltpu
from jax.experimental.pallas import tpu_sc as plsc

_B = 16384
_D = 128
_NC = 2
_NS = 16
_NW = _NC * _NS
_PER_W = _B // _NW
_C = 16
_NSUB = _PER_W // _C
_WT = 5.0


def _body(E_re, E_im, E2_re, E2_im, Rp3, Tp2,
          sv, rv, ov, tv, out,
          idx_s, idx_r, idx_o, idx_t, erows, rrows, trows, out_v, sems):
  cid = lax.axis_index("c")
  sid = lax.axis_index("s")
  wid = sid * _NC + cid
  base = wid * _PER_W

  pltpu.sync_copy(sv.at[pl.ds(base, _PER_W)], idx_s)
  pltpu.sync_copy(rv.at[pl.ds(base, _PER_W)], idx_r)
  pltpu.sync_copy(ov.at[pl.ds(base, _PER_W)], idx_o)
  pltpu.sync_copy(tv.at[pl.ds(base, _PER_W)], idx_t)

  egathers = (
      (E_re, idx_s), (E_im, idx_s),
      (E_re, idx_o), (E_im, idx_o),
      (E2_re, idx_s), (E2_im, idx_s),
      (E2_re, idx_o), (E2_im, idx_o),
  )
  sgathers = (
      (Rp3, idx_r, rrows),
      (Tp2, idx_t, trows),
  )

  lanes = lax.iota(jnp.int32, 16)

  def fire(sub, b):
    sl = pl.ds(sub * _C, _C)
    for k, (tbl, idxf) in enumerate(egathers):
      pltpu.async_copy(tbl.at[idxf.at[sl]], erows.at[b, k], sems.at[b])
    for tbl, idxf, dst in sgathers:
      pltpu.async_copy(tbl.at[idxf.at[sl]], dst.at[b], sems.at[b])

  def wait(sub, b):
    sl = pl.ds(sub * _C, _C)
    for k, (tbl, idxf) in enumerate(egathers):
      pltpu.make_async_copy(tbl.at[idxf.at[sl]], erows.at[b, k],
                            sems.at[b]).wait()
    for tbl, idxf, dst in sgathers:
      pltpu.make_async_copy(tbl.at[idxf.at[sl]], dst.at[b],
                            sems.at[b]).wait()

  def compute(sub, b):
    def dbody(d, accs):
      a0, a1 = accs
      col = jnp.bitwise_and(d + lanes, _D - 1)

      def lde(k):
        return plsc.load_gather(erows.at[b, k], [lanes, col])

      def lds(ref, coloff):
        xi = plsc.load_gather(ref.at[b], [lanes, col + coloff])
        xb = plsc.bitcast(xi, jnp.bfloat16)
        return plsc.unpack(xb, format=plsc.PackFormat.INTERLEAVED)

      esr, esi = lde(0), lde(1)
      eor, eoi = lde(2), lde(3)
      e2sr, e2si = lde(4), lde(5)
      e2or, e2oi = lde(6), lde(7)
      rrr, rri = lds(rrows, 0)
      rsr, rsi = lds(rrows, _D)
      ror, roi = lds(rrows, 2 * _D)
      tsr, tsi = lds(trows, 0)
      tor, toi = lds(trows, _D)
      bre = esr * rrr - esi * rri
      bim = esr * rri + esi * rrr
      term = bre * eor + bim * eoi
      sre = esr * rsr - esi * rsi
      sim = esr * rsi + esi * rsr
      srt = sre * tsr + sim * tsi
      ore = eor * ror - eoi * roi
      oim = eor * roi + eoi * ror
      ort = ore * tor + oim * toi
      tre = e2sr * tsr - e2si * tsi
      tim = e2sr * tsi + e2si * tsr
      sot = tre * e2or + tim * e2oi
      return (a0 + (term + srt), a1 + (ort + _WT * sot))

    z = jnp.zeros((16,), jnp.float32)
    a0, a1 = lax.fori_loop(0, _D, dbody, (z, z), unroll=2)
    out_v[pl.ds(sub * _C, _C)] = a0 + a1

  fire(0, 0)

  def pipe(it, carry):
    sub0 = it * 2
    fire(sub0 + 1, 1)
    wait(sub0, 0)
    compute(sub0, 0)

    @pl.when(sub0 + 2 < _NSUB)
    def _():
      fire(sub0 + 2, 0)

    wait(sub0 + 1, 1)
    compute(sub0 + 1, 1)
    return carry

  lax.fori_loop(0, _NSUB // 2, pipe, 0)
  pltpu.sync_copy(out_v, out.at[pl.ds(base, _PER_W)])


_mesh = plsc.VectorSubcoreMesh(core_axis_name="c", subcore_axis_name="s",
                               num_cores=_NC, num_subcores=_NS)

_score = pl.kernel(
    _body,
    out_type=jax.ShapeDtypeStruct((_B,), jnp.float32),
    mesh=_mesh,
    scratch_types=[
        pltpu.VMEM((_PER_W,), jnp.int32),
        pltpu.VMEM((_PER_W,), jnp.int32),
        pltpu.VMEM((_PER_W,), jnp.int32),
        pltpu.VMEM((_PER_W,), jnp.int32),
        pltpu.VMEM((2, 8, _C, _D), jnp.float32),
        pltpu.VMEM((2, _C, 3 * _D), jnp.int32),
        pltpu.VMEM((2, _C, 2 * _D), jnp.int32),
        pltpu.VMEM((_PER_W,), jnp.float32),
        pltpu.SemaphoreType.DMA((2,)),
    ],
    compiler_params=pltpu.CompilerParams(needs_layout_passes=False),
)


def _pack_bf16_pair(re, im):
  n = re.shape[0]
  both = jnp.stack([re.astype(jnp.bfloat16), im.astype(jnp.bfloat16)],
                   axis=-1)
  return lax.bitcast_convert_type(both, jnp.int32).reshape(n, _D)


@jax.jit
def kernel(E_re, E_im, E2_re, E2_im, R_re, R_im, Rs_re, Rs_im, Ro_re, Ro_im,
           Ts_re, Ts_im, To_re, To_im, s, r, o, t):
  rp3 = jnp.concatenate(
      [_pack_bf16_pair(R_re, R_im),
       _pack_bf16_pair(_WT * Rs_re, _WT * Rs_im),
       _pack_bf16_pair(_WT * Ro_re, _WT * Ro_im)], axis=1)
  tp2 = jnp.concatenate(
      [_pack_bf16_pair(Ts_re, Ts_im),
       _pack_bf16_pair(To_re, To_im)], axis=1)
  return _score(E_re, E_im, E2_re, E2_im, rp3, tp2,
                s.astype(jnp.int32), r.astype(jnp.int32),
                o.astype(jnp.int32), t.astype(jnp.int32))
